# Initial kernel scaffold; baseline (speedup 1.0000x reference)
import jax
import jax.numpy as jnp
from jax.experimental import pallas as pl


def kernel(x, edges, pairs_idx, W_enn, b_enn):
    raise NotImplementedError("write your pallas kernel here")


# speedup vs baseline: 4.7937x; 4.7937x over previous
import functools

import jax
import jax.numpy as jnp
from jax import lax
from jax.experimental import pallas as pl
from jax.experimental.pallas import tpu as pltpu
from jax.experimental.pallas import tpu_sc as plsc

_PAD_VAL = -999.0
_L = 16


def _coef_body(e_ref, w_ref, b_ref, o_ref, *, scale):
    e = e_ref[...]
    a = jnp.dot(e, w_ref[...], preferred_element_type=jnp.float32)
    a = (a + b_ref[0:1, :]) * scale
    mask = e[:, 0:1] == _PAD_VAL
    o_ref[...] = jnp.where(mask, 0.0, a)


def _edge_coefs(edges2d, Wcat, bb, scale, block_e):
    E, DE = edges2d.shape
    KD = Wcat.shape[1]
    return pl.pallas_call(
        functools.partial(_coef_body, scale=scale),
        grid=(E // block_e,),
        in_specs=[
            pl.BlockSpec((block_e, DE), lambda i: (i, 0)),
            pl.BlockSpec((DE, KD), lambda i: (0, 0)),
            pl.BlockSpec((8, KD), lambda i: (0, 0)),
        ],
        out_specs=pl.BlockSpec((block_e, KD), lambda i: (i, 0)),
        out_shape=jax.ShapeDtypeStruct((E, KD), jnp.float32),
    )(edges2d, Wcat, bb)


def _mp_body(a_hbm, x_hbm, p0_hbm, p1_hbm, out_hbm,
             a_v, xg_v, ax_v, idxg_v, idxs_v, stage_v, m_sh, sem,
             *, NPAD, D, K, C, EPT, NCHUNK, RPT, ZR):
    cid = lax.axis_index("c")
    sid = lax.axis_index("s")
    nj = D // _L
    lanes = jnp.arange(_L, dtype=jnp.int32)

    gd = lax.GatherDimensionNumbers(
        offset_dims=(), collapsed_slice_dims=(0,), start_index_map=(0,))

    def tap(e_row, off):
        if 0 <= off and off + _L <= D:
            return xg_v[e_row, pl.ds(off, _L)]
        base = max(0, min(off, D - _L))
        v = xg_v[e_row, pl.ds(base, _L)]
        rel = jnp.clip(lanes + (off - base), 0, _L - 1)
        w = lax.gather(v, rel[:, None], gd, slice_sizes=(1,),
                       mode=lax.GatherScatterMode.PROMISE_IN_BOUNDS)
        pos = lanes + off
        return jnp.where((pos >= 0) & (pos < D), w, 0.0)

    def zrow(i, carry):
        for j in range(nj):
            stage_v[i, pl.ds(_L * j, _L)] = jnp.zeros((_L,), jnp.float32)
        return carry
    lax.fori_loop(0, ZR, zrow, 0)
    base_row = sid * RPT
    for r in range(RPT // ZR):
        pltpu.sync_copy(stage_v, m_sh.at[pl.ds(base_row + r * ZR, ZR)])
    plsc.subcore_barrier()

    ebase = (cid * 16 + sid) * EPT

    def chunk(g, carry):
        b = ebase + g * C
        pltpu.sync_copy(p1_hbm.at[pl.ds(b, C)], idxg_v.at[pl.ds(0, C)])
        pltpu.sync_copy(p0_hbm.at[pl.ds(b, C)], idxg_v.at[pl.ds(C, C)])
        pltpu.sync_copy(p0_hbm.at[pl.ds(b, C)], idxs_v.at[pl.ds(0, C)])
        pltpu.sync_copy(p1_hbm.at[pl.ds(b, C)], idxs_v.at[pl.ds(C, C)])
        pltpu.sync_copy(a_hbm.at[pl.ds(b, C)], a_v)
        pltpu.async_copy(x_hbm.at[idxg_v], xg_v, sem).wait()

        def edge(e, ecarry):
            for j in range(nj):
                accf = jnp.zeros((_L,), jnp.float32)
                accr = jnp.zeros((_L,), jnp.float32)
                for k in range(K):
                    av = a_v[e, pl.ds(k * D + _L * j, _L)]
                    off = _L * j + k - (K // 2)
                    accf = accf + av * tap(e, off)
                    accr = accr + av * tap(C + e, off)
                ax_v[e, pl.ds(_L * j, _L)] = accf
                ax_v[C + e, pl.ds(_L * j, _L)] = accr
            return ecarry
        lax.fori_loop(0, C, edge, 0)

        pltpu.sync_copy(ax_v, m_sh.at[idxs_v], add=True)
        return carry
    lax.fori_loop(0, NCHUNK, chunk, 0)

    plsc.subcore_barrier()
    for r in range(RPT // ZR):
        pltpu.sync_copy(m_sh.at[pl.ds(base_row + r * ZR, ZR)], stage_v)
        pltpu.sync_copy(
            stage_v, out_hbm.at[pl.ds(cid * NPAD + base_row + r * ZR, ZR)])


def _sc_message(A, x2d, p0, p1, NPAD, D, K):
    E = A.shape[0]
    C = 40
    EPT = E // 32
    NCHUNK = EPT // C
    RPT = NPAD // 16
    ZR = 40
    assert EPT * 32 == E and NCHUNK * C == EPT and RPT * 16 == NPAD
    assert (RPT % ZR) == 0 and 2 * C <= 128

    mesh = plsc.VectorSubcoreMesh(core_axis_name="c", subcore_axis_name="s")
    body = functools.partial(
        _mp_body, NPAD=NPAD, D=D, K=K, C=C, EPT=EPT,
        NCHUNK=NCHUNK, RPT=RPT, ZR=ZR)
    kfn = pl.kernel(
        body,
        out_type=jax.ShapeDtypeStruct((2 * NPAD, D), jnp.float32),
        mesh=mesh,
        scratch_types=[
            pltpu.VMEM((C, K * D), jnp.float32),
            pltpu.VMEM((2 * C, D), jnp.float32),
            pltpu.VMEM((2 * C, D), jnp.float32),
            pltpu.VMEM((2 * C,), jnp.int32),
            pltpu.VMEM((2 * C,), jnp.int32),
            pltpu.VMEM((ZR, D), jnp.float32),
            pltpu.VMEM_SHARED((NPAD, D), jnp.float32),
            pltpu.SemaphoreType.DMA,
        ],
    )
    return kfn(A, x2d, p0, p1)


def _add_body(a_ref, b_ref, o_ref):
    o_ref[...] = a_ref[...] + b_ref[...]


def _combine(partials, N, NPAD, D, block_n):
    nb = N // block_n
    off = NPAD // block_n
    return pl.pallas_call(
        _add_body,
        grid=(nb,),
        in_specs=[
            pl.BlockSpec((block_n, D), lambda i: (i, 0)),
            pl.BlockSpec((block_n, D), lambda i, _o=off: (i + _o, 0)),
        ],
        out_specs=pl.BlockSpec((block_n, D), lambda i: (i, 0)),
        out_shape=jax.ShapeDtypeStruct((N, D), jnp.float32),
    )(partials, partials)


def kernel(x, edges, pairs_idx, W_enn, b_enn):
    B, N, D = x.shape
    _, E, DE = edges.shape
    K = W_enn.shape[1] // D
    assert B == 1 and K == 4 and D % _L == 0

    scale = 1.0 / (K ** 0.5)
    edges2d = edges.reshape(E, DE)
    Wcat = W_enn.reshape(DE, D, K).transpose(0, 2, 1).reshape(DE, K * D)
    bcat = b_enn.reshape(D, K).T.reshape(K * D)
    bb = jnp.broadcast_to(bcat.reshape(1, K * D), (8, K * D))
    p0 = pairs_idx[0, :, 0]
    p1 = pairs_idx[0, :, 1]

    NPAD = 10240
    assert N <= NPAD

    A = _edge_coefs(edges2d, Wcat, bb, scale, block_e=2000)
    partials = _sc_message(A, x[0], p0, p1, NPAD, D, K)
    m = _combine(partials, N, NPAD, D, block_n=80)
    return m.reshape(B, N, D)
